# baseline (speedup 1.0000x reference)
import functools

import jax
import jax.numpy as jnp
from jax import lax
from jax.experimental import pallas as pl
from jax.experimental.pallas import tpu as pltpu
from jax.experimental.pallas import tpu_sc as plsc

EMB_DIM = 64
OUT_DIM = 32
LB = 128
NBUF = 10
UB = 528


def _sc_block_gather(table_t, s_idx, order, batch):
    info = plsc.get_sparse_core_info()
    nw = info.num_cores * info.num_subcores
    seg = batch // nw
    mesh = plsc.VectorSubcoreMesh(core_axis_name="c", subcore_axis_name="s")

    @functools.partial(
        pl.kernel,
        mesh=mesh,
        out_type=jax.ShapeDtypeStruct((batch * EMB_DIM,), jnp.float32),
        scratch_types=[
            pltpu.VMEM((seg,), jnp.int32),
            pltpu.VMEM((UB,), jnp.int32),
            pltpu.VMEM((UB,), jnp.int32),
            pltpu.VMEM((seg,), jnp.int32),
            pltpu.VMEM((seg,), jnp.int32),
            pltpu.VMEM((NBUF, EMB_DIM, LB), jnp.float32),
            pltpu.VMEM((seg * EMB_DIM,), jnp.float32),
            pltpu.SemaphoreType.DMA((NBUF,)),
            pltpu.SemaphoreType.DMA,
        ],
        compiler_params=pltpu.CompilerParams(
            use_tc_tiling_on_sc=True, needs_layout_passes=False
        ),
    )
    def k1(table_hbm, sidx_hbm, ord_hbm, out_hbm,
           sidx_v, ublk_v, fj_v, lane_v, ord_v, ring_v, rows_v, sems, sem_w):
        wid = lax.axis_index("s") * info.num_cores + lax.axis_index("c")
        pltpu.sync_copy(sidx_hbm.at[pl.ds(wid * seg, seg)], sidx_v)
        pltpu.sync_copy(ord_hbm.at[pl.ds(wid * seg, seg)], ord_v)

        lanes16 = lax.iota(jnp.int32, 16)
        shift_idx = jnp.maximum(lanes16 - 1, 0)

        carry = jnp.int32(0)
        prev_last = jnp.int32(-1)
        for g in range(seg // 16):
            v = sidx_v[pl.ds(16 * g, 16)]
            b = lax.shift_right_logical(v, 7)
            lane_v[pl.ds(16 * g, 16)] = jnp.bitwise_and(v, LB - 1)
            prevvec = jnp.where(
                lanes16 == 0,
                jnp.broadcast_to(prev_last, (16,)),
                jnp.take(b, shift_idx),
            )
            is_new = b != prevvec
            cnt = jnp.cumsum(is_new.astype(jnp.int32))
            bpos = cnt - 1 + carry
            plsc.store_scatter(ublk_v, [bpos], b, mask=is_new)
            plsc.store_scatter(fj_v, [bpos], lanes16 + 16 * g, mask=is_new)
            carry = carry + jnp.max(cnt)
            prev_last = jnp.max(b)
        n_u = carry
        plsc.store_scatter(
            fj_v,
            [jnp.broadcast_to(n_u, (16,))],
            jnp.full((16,), seg, jnp.int32),
            mask=lanes16 == 0,
        )

        def sread(vec_ref, i):
            g = vec_ref[pl.ds(pl.multiple_of(jnp.bitwise_and(i, -16), 16), 16)]
            return jnp.sum(jnp.where(lanes16 == jnp.bitwise_and(i, 15), g, 0))

        def fetch(u, slot):
            @pl.when(u < n_u)
            def _():
                off = pl.multiple_of(sread(ublk_v, u) * LB, LB)
                pltpu.make_async_copy(
                    table_hbm.at[:, pl.ds(off, LB)],
                    ring_v.at[slot],
                    sems.at[slot],
                ).start()

        for p in range(NBUF):
            fetch(p, p)

        def extract(j, slot):
            lanev = jnp.broadcast_to(sread(lane_v, j), (16,))
            for k in range(EMB_DIM // 16):
                vals = plsc.load_gather(
                    ring_v.at[slot],
                    [lanes16 + 16 * k, lanev],
                )
                rows_v[pl.ds(j * EMB_DIM + 16 * k, 16)] = vals
            dst = pl.multiple_of(sread(ord_v, j) * EMB_DIM, EMB_DIM)
            pltpu.make_async_copy(
                rows_v.at[pl.ds(pl.multiple_of(j * EMB_DIM, EMB_DIM), EMB_DIM)],
                out_hbm.at[pl.ds(dst, EMB_DIM)],
                sem_w,
            ).start()
            return j + 1

        def chunk(g, c):
            for p in range(NBUF):
                u = g * NBUF + p

                @pl.when(u < n_u)
                def _(u=u, p=p):
                    pltpu.make_async_copy(
                        table_hbm.at[:, pl.ds(0, LB)], ring_v.at[p], sems.at[p]
                    ).wait()
                    lax.fori_loop(sread(fj_v, u), sread(fj_v, u + 1),
                                  lambda j, cc, p=p: extract(j, p), 0)
                    fetch(u + NBUF, p)
            return c

        lax.fori_loop(0, (UB + NBUF - 1) // NBUF, chunk, 0)
        pltpu.make_async_copy(
            out_hbm.at[pl.ds(0, seg * EMB_DIM)], rows_v, sem_w
        ).wait()

    return k1(table_t, s_idx, order)


def _tc_mlp_t(emb, W1, b1, W2, b2, batch):
    tb = 4096

    def mlp_kernel(emb_ref, w1_ref, b1_ref, w2_ref, b2_ref, out_ref):
        h = jnp.dot(emb_ref[...], w1_ref[...], preferred_element_type=jnp.float32)
        h = jnp.maximum(h + b1_ref[...], 0.0)
        o_t = lax.dot_general(
            w2_ref[...], h, (((0,), (1,)), ((), ())),
            preferred_element_type=jnp.float32,
        )
        out_ref[...] = o_t + b2_ref[...]

    return pl.pallas_call(
        mlp_kernel,
        grid=(batch // tb,),
        in_specs=[
            pl.BlockSpec((tb, EMB_DIM), lambda i: (i, 0)),
            pl.BlockSpec((EMB_DIM, EMB_DIM), lambda i: (0, 0)),
            pl.BlockSpec((1, EMB_DIM), lambda i: (0, 0)),
            pl.BlockSpec((EMB_DIM, OUT_DIM), lambda i: (0, 0)),
            pl.BlockSpec((OUT_DIM, 1), lambda i: (0, 0)),
        ],
        out_specs=pl.BlockSpec((OUT_DIM, tb), lambda i: (0, i)),
        out_shape=jax.ShapeDtypeStruct((OUT_DIM, batch), jnp.float32),
    )(emb, W1, b1.reshape(1, EMB_DIM), W2, b2.reshape(OUT_DIM, 1))


def kernel(user_id, table, W1, b1, W2, b2):
    batch = user_id.shape[0]
    uid = user_id.astype(jnp.int32)
    s_idx, order = lax.sort_key_val(uid, lax.iota(jnp.int32, batch))
    emb = _sc_block_gather(table.T, s_idx, order, batch)
    out_t = _tc_mlp_t(emb.reshape(batch, EMB_DIM), W1, b1, W2, b2, batch)
    return out_t.T


# speedup vs baseline: 1.0046x; 1.0046x over previous
import functools

import jax
import jax.numpy as jnp
from jax import lax
from jax.experimental import pallas as pl
from jax.experimental.pallas import tpu as pltpu
from jax.experimental.pallas import tpu_sc as plsc

EMB_DIM = 64
OUT_DIM = 32
LB = 128
NBUF = 8
UB = 528


def _sc_block_gather(table_t, s_idx, order, batch):
    info = plsc.get_sparse_core_info()
    nw = info.num_cores * info.num_subcores
    seg = batch // nw
    mesh = plsc.VectorSubcoreMesh(core_axis_name="c", subcore_axis_name="s")

    @functools.partial(
        pl.kernel,
        mesh=mesh,
        out_type=jax.ShapeDtypeStruct((batch * EMB_DIM,), jnp.float32),
        scratch_types=[
            pltpu.VMEM((seg,), jnp.int32),
            pltpu.VMEM((UB,), jnp.int32),
            pltpu.VMEM((UB,), jnp.int32),
            pltpu.VMEM((seg,), jnp.int32),
            pltpu.VMEM((seg,), jnp.int32),
            pltpu.VMEM((NBUF, EMB_DIM, LB), jnp.float32),
            pltpu.VMEM((seg * EMB_DIM,), jnp.float32),
            pltpu.SemaphoreType.DMA((NBUF,)),
            pltpu.SemaphoreType.DMA,
        ],
        compiler_params=pltpu.CompilerParams(
            use_tc_tiling_on_sc=True, needs_layout_passes=False
        ),
    )
    def k1(table_hbm, sidx_hbm, ord_hbm, out_hbm,
           sidx_v, ublk_v, fj_v, lane_v, ord_v, ring_v, rows_v, sems, sem_w):
        wid = lax.axis_index("s") * info.num_cores + lax.axis_index("c")
        pltpu.sync_copy(sidx_hbm.at[pl.ds(wid * seg, seg)], sidx_v)
        pltpu.sync_copy(ord_hbm.at[pl.ds(wid * seg, seg)], ord_v)

        lanes16 = lax.iota(jnp.int32, 16)
        shift_idx = jnp.maximum(lanes16 - 1, 0)

        carry = jnp.int32(0)
        prev_last = jnp.int32(-1)
        for g in range(seg // 16):
            v = sidx_v[pl.ds(16 * g, 16)]
            b = lax.shift_right_logical(v, 7)
            lane_v[pl.ds(16 * g, 16)] = jnp.bitwise_and(v, LB - 1)
            prevvec = jnp.where(
                lanes16 == 0,
                jnp.broadcast_to(prev_last, (16,)),
                jnp.take(b, shift_idx),
            )
            is_new = b != prevvec
            cnt = jnp.cumsum(is_new.astype(jnp.int32))
            bpos = cnt - 1 + carry
            plsc.store_scatter(ublk_v, [bpos], b, mask=is_new)
            plsc.store_scatter(fj_v, [bpos], lanes16 + 16 * g, mask=is_new)
            carry = carry + jnp.max(cnt)
            prev_last = jnp.max(b)
        n_u = carry
        plsc.store_scatter(
            fj_v,
            [jnp.broadcast_to(n_u, (16,))],
            jnp.full((16,), seg, jnp.int32),
            mask=lanes16 == 0,
        )

        def sread(vec_ref, i):
            g = vec_ref[pl.ds(pl.multiple_of(jnp.bitwise_and(i, -16), 16), 16)]
            return jnp.sum(jnp.where(lanes16 == jnp.bitwise_and(i, 15), g, 0))

        def fetch(u, slot):
            @pl.when(u < n_u)
            def _():
                off = pl.multiple_of(sread(ublk_v, u) * LB, LB)
                pltpu.make_async_copy(
                    table_hbm.at[:, pl.ds(off, LB)],
                    ring_v.at[slot],
                    sems.at[slot],
                ).start()

        for p in range(NBUF):
            fetch(p, p)

        def extract(j, slot):
            lanev = jnp.broadcast_to(sread(lane_v, j), (16,))
            for k in range(EMB_DIM // 16):
                vals = plsc.load_gather(
                    ring_v.at[slot],
                    [lanes16 + 16 * k, lanev],
                )
                rows_v[pl.ds(j * EMB_DIM + 16 * k, 16)] = vals
            dst = pl.multiple_of(sread(ord_v, j) * EMB_DIM, EMB_DIM)
            pltpu.make_async_copy(
                rows_v.at[pl.ds(pl.multiple_of(j * EMB_DIM, EMB_DIM), EMB_DIM)],
                out_hbm.at[pl.ds(dst, EMB_DIM)],
                sem_w,
            ).start()
            return j + 1

        def chunk(g, c):
            for p in range(NBUF):
                u = g * NBUF + p

                @pl.when(u < n_u)
                def _(u=u, p=p):
                    pltpu.make_async_copy(
                        table_hbm.at[:, pl.ds(0, LB)], ring_v.at[p], sems.at[p]
                    ).wait()
                    lax.fori_loop(sread(fj_v, u), sread(fj_v, u + 1),
                                  lambda j, cc, p=p: extract(j, p), 0)
                    fetch(u + NBUF, p)
            return c

        lax.fori_loop(0, lax.shift_right_logical(n_u + NBUF - 1, 3), chunk, 0)
        pltpu.make_async_copy(
            out_hbm.at[pl.ds(0, seg * EMB_DIM)], rows_v, sem_w
        ).wait()

    return k1(table_t, s_idx, order)


def _tc_mlp_t(emb, W1, b1, W2, b2, batch):
    tb = 2048

    def mlp_kernel(emb_ref, w1_ref, b1_ref, w2_ref, b2_ref, out_ref):
        h = jnp.dot(emb_ref[...], w1_ref[...], preferred_element_type=jnp.float32)
        h = jnp.maximum(h + b1_ref[...], 0.0)
        o_t = lax.dot_general(
            w2_ref[...], h, (((0,), (1,)), ((), ())),
            preferred_element_type=jnp.float32,
        )
        out_ref[...] = o_t + b2_ref[...]

    return pl.pallas_call(
        mlp_kernel,
        grid=(batch // tb,),
        in_specs=[
            pl.BlockSpec((tb, EMB_DIM), lambda i: (i, 0)),
            pl.BlockSpec((EMB_DIM, EMB_DIM), lambda i: (0, 0)),
            pl.BlockSpec((1, EMB_DIM), lambda i: (0, 0)),
            pl.BlockSpec((EMB_DIM, OUT_DIM), lambda i: (0, 0)),
            pl.BlockSpec((OUT_DIM, 1), lambda i: (0, 0)),
        ],
        out_specs=pl.BlockSpec((OUT_DIM, tb), lambda i: (0, i)),
        out_shape=jax.ShapeDtypeStruct((OUT_DIM, batch), jnp.float32),
    )(emb, W1, b1.reshape(1, EMB_DIM), W2, b2.reshape(OUT_DIM, 1))


def kernel(user_id, table, W1, b1, W2, b2):
    batch = user_id.shape[0]
    uid = user_id.astype(jnp.int32)
    s_idx, order = lax.sort_key_val(uid, lax.iota(jnp.int32, batch))
    emb = _sc_block_gather(table.T, s_idx, order, batch)
    out_t = _tc_mlp_t(emb.reshape(batch, EMB_DIM), W1, b1, W2, b2, batch)
    return out_t.T
